# baseline (speedup 1.0000x reference)
import functools

import jax
import jax.numpy as jnp
from jax import lax
from jax.experimental import pallas as pl
from jax.experimental.pallas import tpu as pltpu
from jax.experimental.pallas import tpu_sc as plsc

NUM_TRACKS = 1000000
EMBED_DIM = 64
BATCH = 16384

_info = plsc.get_sparse_core_info()
_NC, _NS, _L = _info.num_cores, _info.num_subcores, _info.num_lanes
_NW = _NC * _NS
_B_PER_W = BATCH // _NW
_CHUNK = 128
_NCHUNK = _B_PER_W // _CHUNK
_VREGS_PER_ROW = EMBED_DIM // _L


def _gmf_body(ids_hbm, table_hbm, user_hbm, out_hbm,
              idx_v, rows_v, user_v, *sems):
    wid = lax.axis_index("s") * _NC + lax.axis_index("c")
    base = wid * _B_PER_W

    pltpu.sync_copy(user_hbm.at[0], user_v)
    for j in range(_NCHUNK):
        pltpu.sync_copy(
            ids_hbm.at[pl.ds(base + j * _CHUNK, _CHUNK)], idx_v.at[j])

    copies = []
    for j in range(_NCHUNK):
        copies.append(pltpu.async_copy(
            table_hbm.at[idx_v.at[j]], rows_v.at[j], sems[j]))

    u = [user_v[pl.ds(c * _L, _L)] for c in range(_VREGS_PER_ROW)]

    for j in range(_NCHUNK):
        copies[j].wait()

        def mul_row(r, carry, j=j):
            for c in range(_VREGS_PER_ROW):
                sl = pl.ds(c * _L, _L)
                rows_v[j, r, sl] = rows_v[j, r, sl] * u[c]
            return carry

        lax.fori_loop(0, _CHUNK, mul_row, 0)
        pltpu.sync_copy(
            rows_v.at[j], out_hbm.at[pl.ds(base + j * _CHUNK, _CHUNK)])


@jax.jit
def _gmf(track_ids, track_embedding, user_embedding):
    mesh = plsc.VectorSubcoreMesh(core_axis_name="c", subcore_axis_name="s")
    run = pl.kernel(
        _gmf_body,
        mesh=mesh,
        out_type=jax.ShapeDtypeStruct((BATCH, EMBED_DIM), jnp.float32),
        scratch_types=[
            pltpu.VMEM((_NCHUNK, _CHUNK), jnp.int32),
            pltpu.VMEM((_NCHUNK, _CHUNK, EMBED_DIM), jnp.float32),
            pltpu.VMEM((EMBED_DIM,), jnp.float32),
        ] + [pltpu.SemaphoreType.DMA] * _NCHUNK,
        compiler_params=pltpu.CompilerParams(use_tc_tiling_on_sc=False),
    )
    return run(track_ids, track_embedding, user_embedding)


def kernel(track_ids, track_embedding, user_embedding):
    return _gmf(track_ids.astype(jnp.int32), track_embedding, user_embedding)


# speedup vs baseline: 1.7246x; 1.7246x over previous
import functools

import jax
import jax.numpy as jnp
from jax import lax
from jax.experimental import pallas as pl
from jax.experimental.pallas import tpu as pltpu
from jax.experimental.pallas import tpu_sc as plsc

NUM_TRACKS = 1000000
EMBED_DIM = 64
BATCH = 16384

_info = plsc.get_sparse_core_info()
_NC, _NS, _L = _info.num_cores, _info.num_subcores, _info.num_lanes
_NW = _NC * _NS
_B_PER_W = BATCH // _NW
_VREGS_PER_ROW = EMBED_DIM // _L


def _gmf_body(ids_hbm, table_hbm, user_hbm, out_hbm,
              ids_v, rows_v, user_v, sem):
    wid = lax.axis_index("s") * _NC + lax.axis_index("c")
    base = wid * _B_PER_W

    pltpu.sync_copy(user_hbm.at[0], user_v)
    pltpu.sync_copy(ids_hbm.at[pl.ds(base, _B_PER_W)], ids_v)

    def fire(g, carry):
        vec = ids_v[pl.ds(g * _L, _L)]
        for k in range(_L):
            t = vec[k]
            r = g * _L + k
            pltpu.async_copy(table_hbm.at[pl.ds(t, 1)],
                             rows_v.at[pl.ds(r, 1)], sem)
        return carry

    lax.fori_loop(0, _B_PER_W // _L, fire, 0)
    pltpu.make_async_copy(table_hbm.at[pl.ds(0, _B_PER_W)], rows_v, sem).wait()

    u = [user_v[pl.ds(c * _L, _L)] for c in range(_VREGS_PER_ROW)]

    def mul_row(r, carry):
        for c in range(_VREGS_PER_ROW):
            sl = pl.ds(c * _L, _L)
            rows_v[r, sl] = rows_v[r, sl] * u[c]
        return carry

    lax.fori_loop(0, _B_PER_W, mul_row, 0)
    pltpu.sync_copy(rows_v, out_hbm.at[pl.ds(base, _B_PER_W)])


@jax.jit
def _gmf(track_ids, track_embedding, user_embedding):
    mesh = plsc.VectorSubcoreMesh(core_axis_name="c", subcore_axis_name="s")
    run = pl.kernel(
        _gmf_body,
        mesh=mesh,
        out_type=jax.ShapeDtypeStruct((BATCH, EMBED_DIM), jnp.float32),
        scratch_types=[
            pltpu.VMEM((_B_PER_W,), jnp.int32),
            pltpu.VMEM((_B_PER_W, EMBED_DIM), jnp.float32),
            pltpu.VMEM((EMBED_DIM,), jnp.float32),
            pltpu.SemaphoreType.DMA,
        ],
        compiler_params=pltpu.CompilerParams(use_tc_tiling_on_sc=True),
    )
    return run(track_ids, track_embedding, user_embedding)


def kernel(track_ids, track_embedding, user_embedding):
    return _gmf(track_ids.astype(jnp.int32), track_embedding, user_embedding)
